# baseline (speedup 1.0000x reference)
import functools

import jax
import jax.numpy as jnp
from jax import lax
from jax.experimental import pallas as pl
from jax.experimental.pallas import tpu as pltpu
from jax.experimental.pallas import tpu_sc as plsc

_CH = 32
_NBUF = 4


def _sc_gather(table, idx3):
    _, c, d = table.shape
    info = plsc.get_sparse_core_info()
    nc, ns = info.num_cores, info.num_subcores
    nw = nc * ns
    nch = idx3.shape[1]
    b = nw * nch * _CH
    mesh = plsc.VectorSubcoreMesh(core_axis_name="c", subcore_axis_name="s")

    @functools.partial(
        pl.kernel,
        mesh=mesh,
        out_type=jax.ShapeDtypeStruct((b, c, d), jnp.float32),
        scratch_types=[
            pltpu.VMEM((nch, _CH), jnp.int32),
            pltpu.VMEM((_CH, c, d), jnp.float32),
            pltpu.SemaphoreType.DMA,
        ],
    )
    def k(table_hbm, idx_hbm, out_hbm, idx_v, rows_v, sem):
        wid = lax.axis_index("s") * nc + lax.axis_index("c")
        base = wid * (nch * _CH)
        pltpu.sync_copy(idx_hbm.at[wid], idx_v)
        for j in range(nch):
            pltpu.async_copy(table_hbm.at[idx_v.at[j]], rows_v, sem).wait()
            pltpu.sync_copy(rows_v, out_hbm.at[pl.ds(base + j * _CH, _CH)])

    return k(table, idx3)


def _tc_assemble(cls, prefix, suffix, br):
    b, c, d = cls.shape
    p = prefix.shape[1]
    s = suffix.shape[1]
    seq = p + c + s
    nb = b // br

    def body(cls_ref, pre_ref, suf_ref, out_ref, obuf, sems):
        i = pl.program_id(0)
        k = lax.rem(i, _NBUF)

        def slot_copy(slot, block):
            return pltpu.make_async_copy(
                obuf.at[slot], out_ref.at[pl.ds(block * br, br)],
                sems.at[slot])

        @pl.when(i >= _NBUF)
        def _():
            slot_copy(k, i - _NBUF).wait()

        obuf[k, :, 0:p, :] = jnp.broadcast_to(pre_ref[...], (br, p, d))
        obuf[k, :, p:p + c, :] = cls_ref[...]
        obuf[k, :, p + c:seq, :] = jnp.broadcast_to(suf_ref[...], (br, s, d))
        slot_copy(k, i).start()

        @pl.when(i == nb - 1)
        def _():
            for j in range(_NBUF):
                slot_copy(j, 0).wait()

    return pl.pallas_call(
        body,
        grid=(nb,),
        in_specs=[
            pl.BlockSpec((br, c, d), lambda i: (i, 0, 0)),
            pl.BlockSpec(memory_space=pltpu.VMEM),
            pl.BlockSpec(memory_space=pltpu.VMEM),
        ],
        out_specs=pl.BlockSpec(memory_space=pl.ANY),
        out_shape=jax.ShapeDtypeStruct((b, seq, d), jnp.float32),
        scratch_shapes=[
            pltpu.VMEM((_NBUF, br, seq, d), jnp.float32),
            pltpu.SemaphoreType.DMA((_NBUF,)),
        ],
        compiler_params=pltpu.CompilerParams(
            dimension_semantics=("arbitrary",),
        ),
    )(cls, prefix, suffix)


def kernel(label, cls_ctx, token_prefix, token_suffix):
    b = label.shape[0]

    info = plsc.get_sparse_core_info()
    nw = info.num_cores * info.num_subcores
    nch = b // (nw * _CH)

    idx3 = label.astype(jnp.int32).reshape(nw, nch, _CH)
    cls = _sc_gather(cls_ctx, idx3)
    return _tc_assemble(cls, token_prefix, token_suffix, br=64)
